# baseline (speedup 1.0000x reference)
import functools

import jax
import jax.numpy as jnp
from jax import lax
from jax.experimental import pallas as pl
from jax.experimental.pallas import tpu as pltpu

_TN = 256


def _build_u(xrep_ref, b_ref, u_ref):
    xr = xrep_ref[0]
    rows = lax.broadcasted_iota(jnp.int32, (xr.shape[0], 1), 0)
    u = jnp.zeros((xr.shape[0], 32), jnp.float32)
    for j in range(3):
        uj = jnp.dot(xr, b_ref[j], preferred_element_type=jnp.float32)
        u = jnp.where(rows % 3 == j, uj, u)
    u_ref[...] = u.astype(u_ref.dtype)


def _layer_body(relu_first, emit_bf16, w_ref, xrep_ref, b_ref, bias_ref,
                mask_ref, *refs):
    if emit_bf16:
        o_ref, wb_ref, u_ref = refs
    else:
        o_ref, u_ref = refs
        wb_ref = None

    @pl.when(pl.program_id(1) == 0)
    def _():
        _build_u(xrep_ref, b_ref, u_ref)

    w = w_ref[0]
    if emit_bf16:
        wb_ref[0] = w.astype(jnp.bfloat16)
    z = jnp.dot(w, u_ref[...], preferred_element_type=jnp.float32)
    z = z + bias_ref[...]
    if relu_first:
        z = jnp.concatenate([jnp.maximum(z[:, :16], 0.0), z[:, 16:]], axis=1)
    o_ref[0] = z * mask_ref[0]


def _layer(w_in, xrep, b_mat, bias, mask, relu_first, emit_bf16):
    bs, n, nj = w_in.shape
    dcur = xrep.shape[-1]
    grid = (bs, n // _TN)
    out_shape = [jax.ShapeDtypeStruct((bs, n, 32), jnp.float32)]
    out_specs = [pl.BlockSpec((1, _TN, 32), lambda b, t: (b, t, 0))]
    if emit_bf16:
        out_shape.append(jax.ShapeDtypeStruct((bs, n, nj), jnp.bfloat16))
        out_specs.append(pl.BlockSpec((1, _TN, nj), lambda b, t: (b, t, 0)))
    udtype = jnp.bfloat16 if w_in.dtype == jnp.bfloat16 else jnp.float32
    return pl.pallas_call(
        functools.partial(_layer_body, relu_first, emit_bf16),
        grid=grid,
        in_specs=[
            pl.BlockSpec((1, _TN, nj), lambda b, t: (b, t, 0)),
            pl.BlockSpec((1, nj, dcur), lambda b, t: (b, 0, 0)),
            pl.BlockSpec((3, dcur, 32), lambda b, t: (0, 0, 0)),
            pl.BlockSpec((1, 32), lambda b, t: (0, 0)),
            pl.BlockSpec((1, _TN, 1), lambda b, t: (b, t, 0)),
        ],
        out_specs=out_specs,
        out_shape=out_shape,
        scratch_shapes=[pltpu.VMEM((nj, 32), udtype)],
    )(w_in, xrep, b_mat, bias, mask)


def _fold(w1, w2, dcur):
    wcat = jnp.concatenate([w1, w2], axis=0)
    return wcat.reshape(32, 3, dcur).transpose(1, 2, 0)


def kernel(W, x, mask, N_batch, fc1_w0, fc1_b0, fc2_w0, fc2_b0, fc1_w1, fc1_b1,
           fc2_w1, fc2_b1, fc1_w2, fc1_b2, fc2_w2, fc2_b2, fcl_w, fcl_b):
    bs, n = W.shape[0], W.shape[1]
    w_flat = W.reshape(bs, n, n * 3)

    b0 = _fold(fc1_w0, fc2_w0, 8)
    b1 = _fold(fc1_w1, fc2_w1, 32)
    b2 = _fold(fc1_w2, fc2_w2, 32)
    wc3 = jnp.zeros((32, 96), jnp.float32).at[:2].set(fcl_w)
    b3 = wc3.reshape(32, 3, 32).transpose(1, 2, 0)

    bias0 = jnp.concatenate([fc1_b0, fc2_b0])[None]
    bias1 = jnp.concatenate([fc1_b1, fc2_b1])[None]
    bias2 = jnp.concatenate([fc1_b2, fc2_b2])[None]
    bias3 = jnp.zeros((1, 32), jnp.float32).at[0, :2].set(fcl_b)

    cur, w_b16 = _layer(w_flat, jnp.repeat(x, 3, axis=1), b0, bias0, mask,
                        True, True)
    for b_mat, bias, relu_first in ((b1, bias1, True), (b2, bias2, True),
                                    (b3, bias3, False)):
        (cur,) = _layer(w_b16, jnp.repeat(cur, 3, axis=1), b_mat, bias, mask,
                        relu_first, False)
    return cur[:, :, :2]


# speedup vs baseline: 1.3732x; 1.3732x over previous
import functools

import jax
import jax.numpy as jnp
from jax.experimental import pallas as pl
from jax.experimental.pallas import tpu as pltpu

_TN = 256


def _layer_body(relu_first, emit_bf16, n, w_ref, x_ref, b_ref, bias_ref,
                mask_ref, *refs):
    if emit_bf16:
        o_ref, wb_ref, u_ref = refs
    else:
        o_ref, u_ref = refs

    @pl.when(pl.program_id(1) == 0)
    def _():
        xb = x_ref[0]
        for j in range(3):
            u_ref[j] = jnp.dot(xb, b_ref[j],
                               preferred_element_type=jnp.float32
                               ).astype(u_ref.dtype)

    w = w_ref[0]
    if emit_bf16:
        wb_ref[0] = w.astype(jnp.bfloat16)
    z = jnp.dot(w[:, :n], u_ref[0], preferred_element_type=jnp.float32)
    z += jnp.dot(w[:, n:2 * n], u_ref[1], preferred_element_type=jnp.float32)
    z += jnp.dot(w[:, 2 * n:], u_ref[2], preferred_element_type=jnp.float32)
    z = z + bias_ref[...]
    if relu_first:
        z = jnp.concatenate([jnp.maximum(z[:, :16], 0.0), z[:, 16:]], axis=1)
    o_ref[0] = z * mask_ref[0]


def _layer(w_in, xin, b_mat, bias, mask, relu_first, emit_bf16):
    bs, n, dcur = xin.shape
    grid = (bs, n // _TN)
    out_shape = [jax.ShapeDtypeStruct((bs, n, 32), jnp.float32)]
    out_specs = [pl.BlockSpec((1, _TN, 32), lambda b, t: (b, t, 0))]
    if emit_bf16:
        out_shape.append(jax.ShapeDtypeStruct((bs, n, 3 * n), jnp.bfloat16))
        out_specs.append(pl.BlockSpec((1, _TN, 3 * n), lambda b, t: (b, t, 0)))
    udtype = jnp.bfloat16 if w_in.dtype == jnp.bfloat16 else jnp.float32
    out = pl.pallas_call(
        functools.partial(_layer_body, relu_first, emit_bf16, n),
        grid=grid,
        in_specs=[
            pl.BlockSpec((1, _TN, 3 * n), lambda b, t: (b, t, 0)),
            pl.BlockSpec((1, n, dcur), lambda b, t: (b, 0, 0)),
            pl.BlockSpec((3, dcur, 32), lambda b, t: (0, 0, 0)),
            pl.BlockSpec((1, 32), lambda b, t: (0, 0)),
            pl.BlockSpec((1, _TN, 1), lambda b, t: (b, t, 0)),
        ],
        out_specs=out_specs,
        out_shape=out_shape,
        scratch_shapes=[pltpu.VMEM((3, n, 32), udtype)],
    )(w_in, xin, b_mat, bias, mask)
    return out


def _fold(w1, w2, dcur):
    wcat = jnp.concatenate([w1, w2], axis=0)
    return wcat.reshape(32, 3, dcur).transpose(1, 2, 0)


def kernel(W, x, mask, N_batch, fc1_w0, fc1_b0, fc2_w0, fc2_b0, fc1_w1, fc1_b1,
           fc2_w1, fc2_b1, fc1_w2, fc1_b2, fc2_w2, fc2_b2, fcl_w, fcl_b):
    bs, n = W.shape[0], W.shape[1]
    w_flat = W.swapaxes(2, 3).reshape(bs, n, 3 * n)

    b0 = _fold(fc1_w0, fc2_w0, 8)
    b1 = _fold(fc1_w1, fc2_w1, 32)
    b2 = _fold(fc1_w2, fc2_w2, 32)
    wc3 = jnp.zeros((32, 96), jnp.float32).at[:2].set(fcl_w)
    b3 = wc3.reshape(32, 3, 32).transpose(1, 2, 0)

    bias0 = jnp.concatenate([fc1_b0, fc2_b0])[None]
    bias1 = jnp.concatenate([fc1_b1, fc2_b1])[None]
    bias2 = jnp.concatenate([fc1_b2, fc2_b2])[None]
    bias3 = jnp.zeros((1, 32), jnp.float32).at[0, :2].set(fcl_b)

    cur, w_b16 = _layer(w_flat, x, b0, bias0, mask, True, True)
    for b_mat, bias, relu_first in ((b1, bias1, True), (b2, bias2, True),
                                    (b3, bias3, False)):
        (cur,) = _layer(w_b16, cur, b_mat, bias, mask, relu_first, False)
    return cur[:, :, :2]


# speedup vs baseline: 2.8824x; 2.0991x over previous
import functools

import jax
import jax.numpy as jnp
from jax.experimental import pallas as pl
from jax.experimental.pallas import tpu as pltpu

_TN = 256


def _layer_body(relu_first, emit_bf16, w_ref, x_ref, b_ref, bias_ref,
                mask_ref, *refs):
    if emit_bf16:
        o_ref, wb_ref, u_ref = refs
    else:
        o_ref, u_ref = refs

    @pl.when(pl.program_id(1) == 0)
    def _():
        xb = x_ref[0]
        for j in range(3):
            u_ref[j] = jnp.dot(xb, b_ref[j],
                               preferred_element_type=jnp.float32
                               ).astype(u_ref.dtype)

    w = w_ref[0]
    if emit_bf16:
        wb_ref[0] = w.astype(jnp.bfloat16)
    z = jnp.dot(w[0], u_ref[0], preferred_element_type=jnp.float32)
    z += jnp.dot(w[1], u_ref[1], preferred_element_type=jnp.float32)
    z += jnp.dot(w[2], u_ref[2], preferred_element_type=jnp.float32)
    z = z + bias_ref[...]
    if relu_first:
        z = jnp.concatenate([jnp.maximum(z[:, :16], 0.0), z[:, 16:]], axis=1)
    o_ref[0] = z * mask_ref[0]


def _layer(w_in, xin, b_mat, bias, mask, relu_first, emit_bf16):
    bs, n, dcur = xin.shape
    grid = (bs, n // _TN)
    out_shape = [jax.ShapeDtypeStruct((bs, n, 32), jnp.float32)]
    out_specs = [pl.BlockSpec((1, _TN, 32), lambda b, t: (b, t, 0))]
    if emit_bf16:
        out_shape.append(jax.ShapeDtypeStruct((bs, 3, n, n), jnp.bfloat16))
        out_specs.append(
            pl.BlockSpec((1, 3, _TN, n), lambda b, t: (b, 0, t, 0)))
    udtype = jnp.bfloat16 if w_in.dtype == jnp.bfloat16 else jnp.float32
    return pl.pallas_call(
        functools.partial(_layer_body, relu_first, emit_bf16),
        grid=grid,
        in_specs=[
            pl.BlockSpec((1, 3, _TN, n), lambda b, t: (b, 0, t, 0)),
            pl.BlockSpec((1, n, dcur), lambda b, t: (b, 0, 0)),
            pl.BlockSpec((3, dcur, 32), lambda b, t: (0, 0, 0)),
            pl.BlockSpec((1, 32), lambda b, t: (0, 0)),
            pl.BlockSpec((1, _TN, 1), lambda b, t: (b, t, 0)),
        ],
        out_specs=out_specs,
        out_shape=out_shape,
        scratch_shapes=[pltpu.VMEM((3, n, 32), udtype)],
    )(w_in, xin, b_mat, bias, mask)


def _fold(w1, w2, dcur):
    wcat = jnp.concatenate([w1, w2], axis=0)
    return wcat.reshape(32, 3, dcur).transpose(1, 2, 0)


def kernel(W, x, mask, N_batch, fc1_w0, fc1_b0, fc2_w0, fc2_b0, fc1_w1, fc1_b1,
           fc2_w1, fc2_b1, fc1_w2, fc1_b2, fc2_w2, fc2_b2, fcl_w, fcl_b):
    bs, n = W.shape[0], W.shape[1]
    w_sep = jnp.transpose(W, (0, 3, 1, 2))

    b0 = _fold(fc1_w0, fc2_w0, 8)
    b1 = _fold(fc1_w1, fc2_w1, 32)
    b2 = _fold(fc1_w2, fc2_w2, 32)
    wc3 = jnp.zeros((32, 96), jnp.float32).at[:2].set(fcl_w)
    b3 = wc3.reshape(32, 3, 32).transpose(1, 2, 0)

    bias0 = jnp.concatenate([fc1_b0, fc2_b0])[None]
    bias1 = jnp.concatenate([fc1_b1, fc2_b1])[None]
    bias2 = jnp.concatenate([fc1_b2, fc2_b2])[None]
    bias3 = jnp.zeros((1, 32), jnp.float32).at[0, :2].set(fcl_b)

    cur, w_b16 = _layer(w_sep, x, b0, bias0, mask, True, True)
    for b_mat, bias, relu_first in ((b1, bias1, True), (b2, bias2, True),
                                    (b3, bias3, False)):
        (cur,) = _layer(w_b16, cur, b_mat, bias, mask, relu_first, False)
    return cur[:, :, :2]


# speedup vs baseline: 3.1966x; 1.1090x over previous
import functools

import jax
import jax.numpy as jnp
from jax.experimental import pallas as pl
from jax.experimental.pallas import tpu as pltpu

_TN = 512


def _layer_body(relu_first, emit_bf16, w_ref, x_ref, b_ref, bias_ref,
                mask_ref, *refs):
    if emit_bf16:
        o_ref, wb_ref, u_ref = refs
    else:
        o_ref, u_ref = refs

    @pl.when(pl.program_id(1) == 0)
    def _():
        xb = x_ref[0]
        for j in range(3):
            u_ref[j] = jnp.dot(xb, b_ref[j],
                               preferred_element_type=jnp.float32
                               ).astype(u_ref.dtype)

    w = w_ref[0].astype(jnp.bfloat16)
    if emit_bf16:
        wb_ref[0] = w
    z = jnp.dot(w[0], u_ref[0], preferred_element_type=jnp.float32)
    z += jnp.dot(w[1], u_ref[1], preferred_element_type=jnp.float32)
    z += jnp.dot(w[2], u_ref[2], preferred_element_type=jnp.float32)
    z = z + bias_ref[...]
    if relu_first:
        z = jnp.concatenate([jnp.maximum(z[:, :16], 0.0), z[:, 16:]], axis=1)
    o_ref[0] = z * mask_ref[0]


def _layer(w_in, xin, b_mat, bias, mask, relu_first, emit_bf16):
    bs, n, dcur = xin.shape
    grid = (bs, n // _TN)
    out_shape = [jax.ShapeDtypeStruct((bs, n, 32), jnp.float32)]
    out_specs = [pl.BlockSpec((1, _TN, 32), lambda b, t: (b, t, 0))]
    if emit_bf16:
        out_shape.append(jax.ShapeDtypeStruct((bs, 3, n, n), jnp.bfloat16))
        out_specs.append(
            pl.BlockSpec((1, 3, _TN, n), lambda b, t: (b, 0, t, 0)))
    udtype = jnp.bfloat16
    return pl.pallas_call(
        functools.partial(_layer_body, relu_first, emit_bf16),
        grid=grid,
        in_specs=[
            pl.BlockSpec((1, 3, _TN, n), lambda b, t: (b, 0, t, 0)),
            pl.BlockSpec((1, n, dcur), lambda b, t: (b, 0, 0)),
            pl.BlockSpec((3, dcur, 32), lambda b, t: (0, 0, 0)),
            pl.BlockSpec((1, 32), lambda b, t: (0, 0)),
            pl.BlockSpec((1, _TN, 1), lambda b, t: (b, t, 0)),
        ],
        out_specs=out_specs,
        out_shape=out_shape,
        scratch_shapes=[pltpu.VMEM((3, n, 32), udtype)],
    )(w_in, xin, b_mat, bias, mask)


def _fold(w1, w2, dcur):
    wcat = jnp.concatenate([w1, w2], axis=0)
    return wcat.reshape(32, 3, dcur).transpose(1, 2, 0)


def kernel(W, x, mask, N_batch, fc1_w0, fc1_b0, fc2_w0, fc2_b0, fc1_w1, fc1_b1,
           fc2_w1, fc2_b1, fc1_w2, fc1_b2, fc2_w2, fc2_b2, fcl_w, fcl_b):
    bs, n = W.shape[0], W.shape[1]
    w_sep = jnp.transpose(W, (0, 3, 1, 2))

    b0 = _fold(fc1_w0, fc2_w0, 8)
    b1 = _fold(fc1_w1, fc2_w1, 32)
    b2 = _fold(fc1_w2, fc2_w2, 32)
    wc3 = jnp.zeros((32, 96), jnp.float32).at[:2].set(fcl_w)
    b3 = wc3.reshape(32, 3, 32).transpose(1, 2, 0)

    bias0 = jnp.concatenate([fc1_b0, fc2_b0])[None]
    bias1 = jnp.concatenate([fc1_b1, fc2_b1])[None]
    bias2 = jnp.concatenate([fc1_b2, fc2_b2])[None]
    bias3 = jnp.zeros((1, 32), jnp.float32).at[0, :2].set(fcl_b)

    cur, w_b16 = _layer(w_sep, x, b0, bias0, mask, True, True)
    for b_mat, bias, relu_first in ((b1, bias1, True), (b2, bias2, True),
                                    (b3, bias3, False)):
        (cur,) = _layer(w_b16, cur, b_mat, bias, mask, relu_first, False)
    return cur[:, :, :2]
